# baseline (speedup 1.0000x reference)
import functools

import jax
import jax.numpy as jnp
from jax import lax
from jax.experimental import pallas as pl
from jax.experimental.pallas import tpu as pltpu, tpu_sc as plsc


_INFO = plsc.get_sparse_core_info()
_NC, _NS = _INFO.num_cores, _INFO.num_subcores
_NW = _NC * _NS


def _make_lookup(batch, hidden):
    b_per_w = batch // _NW
    mesh = plsc.VectorSubcoreMesh(core_axis_name="c", subcore_axis_name="s")

    @functools.partial(
        pl.kernel,
        mesh=mesh,
        out_type=jax.ShapeDtypeStruct((batch, hidden), jnp.float32),
        scratch_types=[
            pltpu.VMEM((b_per_w,), jnp.int32),
            pltpu.VMEM((b_per_w, hidden), jnp.float32),
            pltpu.SemaphoreType.DMA,
        ],
    )
    def lookup(table_hbm, idx_hbm, out_hbm, idx_v, rows_v, sem):
        wid = lax.axis_index("s") * _NC + lax.axis_index("c")
        base = wid * b_per_w
        pltpu.sync_copy(idx_hbm.at[pl.ds(base, b_per_w)], idx_v)

        def body(g, _):
            gbase = g * 16
            vec = idx_v[pl.ds(gbase, 16)]
            for j in range(16):
                r = vec[j]
                pltpu.async_copy(
                    table_hbm.at[pl.ds(r, 1)],
                    rows_v.at[pl.ds(gbase + j, 1)],
                    sem,
                )
            for j in range(16):
                pltpu.make_async_copy(
                    table_hbm.at[pl.ds(vec[j], 1)],
                    rows_v.at[pl.ds(gbase + j, 1)],
                    sem,
                ).wait()
            return 0

        lax.fori_loop(0, b_per_w // 16, body, 0)
        pltpu.sync_copy(rows_v, out_hbm.at[pl.ds(base, b_per_w)])

    return lookup


def kernel(labels, embedding_table):
    batch = labels.shape[0]
    hidden = embedding_table.shape[1]
    lookup = _make_lookup(batch, hidden)
    return lookup(embedding_table, labels.astype(jnp.int32))


# speedup vs baseline: 1.7486x; 1.7486x over previous
import functools

import jax
import jax.numpy as jnp
from jax import lax
from jax.experimental import pallas as pl
from jax.experimental.pallas import tpu as pltpu, tpu_sc as plsc


_INFO = plsc.get_sparse_core_info()
_NC, _NS = _INFO.num_cores, _INFO.num_subcores
_NW = _NC * _NS
_RING = 8


def _make_lookup(batch, hidden):
    b_per_w = batch // _NW
    mesh = plsc.VectorSubcoreMesh(core_axis_name="c", subcore_axis_name="s")

    @functools.partial(
        pl.kernel,
        mesh=mesh,
        out_type=jax.ShapeDtypeStruct((hidden, batch), jnp.float32),
        scratch_types=[
            pltpu.VMEM((b_per_w,), jnp.int32),
            pltpu.VMEM((_RING, hidden, 128), jnp.float32),
            pltpu.VMEM((hidden, b_per_w), jnp.float32),
            pltpu.SemaphoreType.DMA,
        ],
        compiler_params=pltpu.CompilerParams(needs_layout_passes=False),
    )
    def lookup(table_hbm, idx_hbm, out_hbm, idx_v, blocks_v, cols_v, sem):
        wid = lax.axis_index("s") * _NC + lax.axis_index("c")
        base = pl.multiple_of(wid * b_per_w, 128)
        pltpu.sync_copy(idx_hbm.at[pl.ds(base, b_per_w)], idx_v)

        iota16 = lax.broadcasted_iota(jnp.int32, (16,), 0)

        def fire(slot, col128):
            pltpu.async_copy(
                table_hbm.at[:, pl.ds(pl.multiple_of(col128, 128), 128)],
                blocks_v.at[slot],
                sem,
            )

        def drain(slot, col128):
            pltpu.make_async_copy(
                table_hbm.at[:, pl.ds(pl.multiple_of(col128, 128), 128)],
                blocks_v.at[slot],
                sem,
            ).wait()

        def extract(slot, lane, dst_col):
            lvec = jnp.full((16,), lane, jnp.int32)
            svec = jnp.full((16,), slot, jnp.int32)
            dvec = jnp.full((16,), dst_col, jnp.int32)
            for h in range(hidden // 16):
                hvec = h * 16 + iota16
                vals = plsc.load_gather(blocks_v, [svec, hvec, lvec])
                plsc.store_scatter(cols_v, [hvec, dvec], vals)

        def body(g, _):
            gbase = g * 16
            vec = idx_v[pl.ds(gbase, 16)]
            cols = []
            lanes = []
            for j in range(16):
                r = vec[j]
                c = (r // 128) * 128
                cols.append(c)
                lanes.append(r - c)
            for j in range(_RING):
                fire(j, cols[j])
            for j in range(16 - _RING):
                drain(j, cols[j])
                extract(j, lanes[j], gbase + j)
                fire(j, cols[j + _RING])
            for j in range(16 - _RING, 16):
                drain(j - (16 - _RING), cols[j])
                extract(j - (16 - _RING), lanes[j], gbase + j)
            return 0

        lax.fori_loop(0, b_per_w // 16, body, 0)
        pltpu.sync_copy(cols_v, out_hbm.at[:, pl.ds(base, b_per_w)])

    return lookup


def kernel(labels, embedding_table):
    batch = labels.shape[0]
    hidden = embedding_table.shape[1]
    lookup = _make_lookup(batch, hidden)
    out_t = lookup(embedding_table.T, labels.astype(jnp.int32))
    return out_t.T
